# Initial kernel scaffold; baseline (speedup 1.0000x reference)
import jax
import jax.numpy as jnp
from jax.experimental import pallas as pl


def kernel(x, edge_index, precomp_neigh_edge, connection, W_rad1, W1, Wself1, b1, W_rad2, W2, Wself2, b2):
    raise NotImplementedError("write your pallas kernel here")


# speedup vs baseline: 20.4491x; 20.4491x over previous
import functools

import jax
import jax.numpy as jnp
import numpy as np
from jax import lax
from jax.experimental import pallas as pl
from jax.experimental.pallas import tpu as pltpu
from jax.experimental.pallas import tpu_sc as plsc

_HI = lax.Precision.HIGHEST
_SC_PARAMS = pltpu.CompilerParams(use_tc_tiling_on_sc=False)

_D = 32
_CHUNK = 128
_NC = 2
_NS = 16
_NW = _NC * _NS
_BLK_E = 2000
_BLK_N = 2000


def _sc_gather(tab, idx):
    e = idx.shape[0]
    nch = e // _CHUNK
    iters = -(-nch // _NW)
    mesh = plsc.VectorSubcoreMesh(core_axis_name="c", subcore_axis_name="s")

    @functools.partial(
        pl.kernel,
        out_type=jax.ShapeDtypeStruct((e, _D), jnp.float32),
        mesh=mesh,
        scratch_types=[
            pltpu.VMEM((_CHUNK,), jnp.int32),
            pltpu.VMEM((_CHUNK, _D), jnp.float32),
            pltpu.SemaphoreType.DMA,
        ],
        compiler_params=_SC_PARAMS,
    )
    def gk(tab_hbm, idx_hbm, out_hbm, idx_v, rows_v, sem):
        wid = lax.axis_index("s") * _NC + lax.axis_index("c")

        def body(k, carry):
            ch = wid + k * _NW

            @pl.when(ch < nch)
            def _():
                base = ch * _CHUNK
                pltpu.sync_copy(idx_hbm.at[pl.ds(base, _CHUNK)], idx_v)
                pltpu.async_copy(tab_hbm.at[idx_v], rows_v, sem).wait()
                pltpu.sync_copy(rows_v, out_hbm.at[pl.ds(base, _CHUNK)])

            return carry

        lax.fori_loop(0, iters, body, 0)

    return gk(tab, idx)


def _sc_scatter(msg, dst, zer):
    e = dst.shape[0]
    n = zer.shape[0] * _NS
    rows_t = n // _NS
    nch = e // _CHUNK
    iters = -(-nch // _NW)
    mesh = plsc.VectorSubcoreMesh(core_axis_name="c", subcore_axis_name="s")

    @functools.partial(
        pl.kernel,
        out_type=jax.ShapeDtypeStruct((_NC, n, _D), jnp.float32),
        mesh=mesh,
        scratch_types=[
            pltpu.VMEM((_CHUNK,), jnp.int32),
            pltpu.VMEM((_CHUNK, _D), jnp.float32),
            pltpu.VMEM_SHARED((n, _D), jnp.float32),
        ],
        compiler_params=_SC_PARAMS,
    )
    def sk(msg_hbm, dst_hbm, zer_hbm, out_hbm, idx_v, msg_v, acc):
        cid = lax.axis_index("c")
        sid = lax.axis_index("s")
        wid = sid * _NC + cid
        row0 = sid * rows_t

        pltpu.sync_copy(zer_hbm, acc.at[pl.ds(row0, rows_t)])
        plsc.subcore_barrier()

        def body(k, carry):
            ch = wid + k * _NW

            @pl.when(ch < nch)
            def _():
                base = ch * _CHUNK
                pltpu.sync_copy(dst_hbm.at[pl.ds(base, _CHUNK)], idx_v)
                pltpu.sync_copy(msg_hbm.at[pl.ds(base, _CHUNK)], msg_v)
                pltpu.sync_copy(msg_v, acc.at[idx_v], add=True)

            return carry

        lax.fori_loop(0, iters, body, 0)
        plsc.subcore_barrier()
        pltpu.sync_copy(acc.at[pl.ds(row0, rows_t)],
                        out_hbm.at[cid, pl.ds(row0, rows_t)])

    return sk(msg, dst, zer)


def _edge_math(xs, th, p8, wrad8, pswap, wcat):
    co = jnp.cos(th)
    si = jnp.sin(th)
    col = lax.broadcasted_iota(jnp.int32, xs.shape, 1)
    o = col % 3
    m1 = jnp.where(o != 0, 1.0, 0.0).astype(xs.dtype)
    sg = jnp.where(o == 1, -1.0, jnp.where(o == 2, 1.0, 0.0)).astype(xs.dtype)
    u = (1.0 - m1) + co * m1
    v = si * sg
    xt = xs * u + jnp.dot(xs, pswap, precision=_HI) * v
    coeff = jnp.dot(p8, wrad8, precision=_HI)
    y = jnp.dot(xt, wcat, precision=_HI)
    return (coeff[:, 0:1] * y[:, 0:32]
            + coeff[:, 1:2] * y[:, 32:64]
            + coeff[:, 2:3] * y[:, 64:96])


def _edge_kernel(xs_ref, cn_ref, p8_ref, wrad_ref, pswap_ref, wcat_ref, o_ref):
    o_ref[...] = _edge_math(xs_ref[...], cn_ref[...], p8_ref[...],
                            wrad_ref[...], pswap_ref[...], wcat_ref[...])


def _edge_call(xs, cn, p8, wrad8, pswap, wcat):
    e = xs.shape[0]
    grid = e // _BLK_E
    full = lambda *shape: pl.BlockSpec(shape, lambda i: (0,) * len(shape))
    return pl.pallas_call(
        _edge_kernel,
        grid=(grid,),
        in_specs=[
            pl.BlockSpec((_BLK_E, _D), lambda i: (i, 0)),
            pl.BlockSpec((_BLK_E, 1), lambda i: (i, 0)),
            pl.BlockSpec((_BLK_E, 8), lambda i: (i, 0)),
            full(8, 8),
            full(_D, _D),
            full(_D, 96),
        ],
        out_specs=pl.BlockSpec((_BLK_E, _D), lambda i: (i, 0)),
        out_shape=jax.ShapeDtypeStruct((e, _D), jnp.float32),
    )(xs, cn, p8, wrad8, pswap, wcat)


def _node_math(p0, p1, xin, res, wself, bv, a1, a2):
    h = p0 + p1 + jnp.dot(xin, wself, precision=_HI) + bv[0:1, :]
    if res is not None:
        h = h + res
    z = jnp.maximum(jnp.dot(h, a1, precision=_HI), 0.0)
    return jnp.dot(z, a2, precision=_HI)


def _combine_kernel_nores(p0_ref, p1_ref, x_ref, ws_ref, bv_ref, a1_ref,
                          a2_ref, o_ref):
    o_ref[...] = _node_math(p0_ref[...], p1_ref[...], x_ref[...], None,
                            ws_ref[...], bv_ref[...], a1_ref[...], a2_ref[...])


def _combine_kernel_res(p0_ref, p1_ref, x_ref, r_ref, ws_ref, bv_ref, a1_ref,
                        a2_ref, o_ref):
    o_ref[...] = _node_math(p0_ref[...], p1_ref[...], x_ref[...], r_ref[...],
                            ws_ref[...], bv_ref[...], a1_ref[...], a2_ref[...])


def _combine_call(p0, p1, xin, res, wself, bv, a1, a2):
    n = xin.shape[0]
    grid = n // _BLK_N
    blk = lambda: pl.BlockSpec((_BLK_N, _D), lambda i: (i, 0))
    full = lambda *shape: pl.BlockSpec(shape, lambda i: (0,) * len(shape))
    args = [p0, p1, xin] + ([] if res is None else [res])
    in_specs = [blk() for _ in args] + [
        full(_D, _D), full(8, _D), full(_D, 40), full(40, _D)]
    body = _combine_kernel_nores if res is None else _combine_kernel_res
    return pl.pallas_call(
        body,
        grid=(grid,),
        in_specs=in_specs,
        out_specs=blk(),
        out_shape=jax.ShapeDtypeStruct((n, _D), jnp.float32),
    )(*args, wself, bv, a1, a2)


def _prep_conv_weights(w_rad, w):
    wrad8 = jnp.zeros((8, 8), jnp.float32)
    for b in range(3):
        for r in range(2):
            wrad8 = wrad8.at[b * 2 + r, b].set(w_rad[b, r])
    wc = jnp.transpose(w, (3, 4, 0, 1, 2)).reshape(24, 3, 24)
    wc = jnp.pad(wc, ((0, 8), (0, 0), (0, 8))).reshape(_D, 96)
    return wrad8, wc


def _prep_self(w_self):
    ws = jnp.transpose(w_self, (2, 3, 0, 1)).reshape(24, 24)
    return jnp.pad(ws, ((0, 8), (0, 8)))


def _prep_bias(bias):
    bv = jnp.zeros((_D,), jnp.float32).at[jnp.arange(8) * 3].set(bias)
    return jnp.broadcast_to(bv, (8, _D))


def _static_mats():
    pswap = np.zeros((_D, _D), np.float32)
    for c in range(8):
        pswap[c * 3 + 2, c * 3 + 1] = 1.0
        pswap[c * 3 + 1, c * 3 + 2] = 1.0
    thetas = 2.0 * np.pi * np.arange(5) / 5
    a = np.stack([np.ones(5), np.cos(thetas), np.sin(thetas)], 1)
    scale = np.array([1.0, 2.0, 2.0], np.float32) / 5.0
    a1 = np.zeros((_D, 40), np.float32)
    a2 = np.zeros((40, _D), np.float32)
    for c in range(8):
        for s in range(5):
            for o in range(3):
                a1[c * 3 + o, c * 5 + s] = a[s, o]
                a2[c * 5 + s, c * 3 + o] = a[s, o] * scale[o]
    return (jnp.asarray(pswap), jnp.asarray(a1), jnp.asarray(a2))


def kernel(x, edge_index, precomp_neigh_edge, connection,
           W_rad1, W1, Wself1, b1, W_rad2, W2, Wself2, b2):
    n, e = x.shape[0], edge_index.shape[0]
    assert n % (_NS * _BLK_N // _BLK_N) == 0 and n % _NS == 0
    assert e % _CHUNK == 0 and e % _BLK_E == 0 and n % _BLK_N == 0

    src = edge_index[:, 0].astype(jnp.int32)
    dst = edge_index[:, 1].astype(jnp.int32)
    xp = jnp.pad(x.reshape(n, 24), ((0, 0), (0, 8)))
    p8 = jnp.pad(precomp_neigh_edge.reshape(e, 6), ((0, 0), (0, 2)))
    cn = connection.reshape(e, 1)
    zer = jnp.zeros((n // _NS, _D), jnp.float32)

    pswap, a1, a2 = _static_mats()
    wrad8_1, wcat1 = _prep_conv_weights(W_rad1, W1)
    wrad8_2, wcat2 = _prep_conv_weights(W_rad2, W2)
    ws1, ws2 = _prep_self(Wself1), _prep_self(Wself2)
    bv1, bv2 = _prep_bias(b1), _prep_bias(b2)

    g1 = _sc_gather(xp, src)
    m1 = _edge_call(g1, cn, p8, wrad8_1, pswap, wcat1)
    pr1 = _sc_scatter(m1, dst, zer)
    h = _combine_call(pr1[0], pr1[1], xp, None, ws1, bv1, a1, a2)

    g2 = _sc_gather(h, src)
    m2 = _edge_call(g2, cn, p8, wrad8_2, pswap, wcat2)
    pr2 = _sc_scatter(m2, dst, zer)
    y = _combine_call(pr2[0], pr2[1], h, xp, ws2, bv2, a1, a2)

    return y[:, :24].reshape(n, 8, 3)
